# baseline (speedup 1.0000x reference)
import functools

import jax
import jax.numpy as jnp
from jax import lax
from jax.experimental import pallas as pl
from jax.experimental.pallas import tpu as pltpu
from jax.experimental.pallas import tpu_sc as plsc

DEC = 96
W = 192
T = 4096
B = 32
NUM = T - 2 * W
PADNUM = 3840
G = 5
NGROUP = PADNUM // (G * 128)
TILEW = (G + 1) * 128

_NC = 2
_NS = 16


def _tc_body(x_ref, dists_ref, kb_ref):
    for j in range(W):
        col = x_ref[:, T - W + j : T - W + j + 1]
        kb_ref[:, j * 128 : (j + 1) * 128] = jnp.broadcast_to(col, (B, 128))

    for g in range(NGROUP):
        base = g * G * 128
        accs = [jnp.zeros((B, 128), jnp.float32) for _ in range(G)]
        for jh, njl in ((0, 128), (1, 64)):
            tile = x_ref[:, pl.ds(base + 128 * jh, TILEW)]

            def body(jl, accs, tile=tile, jh=jh):
                sl = (TILEW - jl) % TILEW
                rolled = pltpu.roll(tile, sl, axis=1)
                kjb = kb_ref[:, pl.ds((128 * jh + jl) * 128, 128)]
                out = []
                for s in range(G):
                    d = rolled[:, s * 128 : (s + 1) * 128] - kjb
                    out.append(accs[s] + d * d)
                return out

            accs = jax.lax.fori_loop(0, njl, body, accs, unroll=4)
        for s in range(G):
            dists_ref[:, base + s * 128 : base + (s + 1) * 128] = accs[s] / W


_sc_mesh = plsc.VectorSubcoreMesh(core_axis_name="c", subcore_axis_name="s")


@functools.partial(
    pl.kernel,
    out_type=jax.ShapeDtypeStruct((B * DEC,), jnp.float32),
    mesh=_sc_mesh,
    scratch_types=[
        pltpu.VMEM((PADNUM,), jnp.float32),
        pltpu.VMEM((T,), jnp.float32),
        pltpu.VMEM((DEC,), jnp.float32),
        pltpu.VMEM((16,), jnp.float32),
        pltpu.VMEM((16,), jnp.int32),
    ],
    compiler_params=pltpu.CompilerParams(needs_layout_passes=False),
)
def _sc_argmin_gather(dists_hbm, x_hbm, out_hbm, d_v, x_v, o_v, tf_v, ti_v):
    b = lax.axis_index("s") * _NC + lax.axis_index("c")
    pltpu.sync_copy(dists_hbm.at[pl.ds(pl.multiple_of(b * PADNUM, 8), PADNUM)], d_v)
    pltpu.sync_copy(x_hbm.at[pl.ds(pl.multiple_of(b * T, 8), T)], x_v)
    lanes = lax.iota(jnp.int32, 16)

    def body(i, carry):
        mv, mi = carry
        v = d_v[pl.ds(i * 16, 16)]
        idx = lanes + i * 16
        p = v < mv
        return jnp.where(p, v, mv), jnp.where(p, idx, mi)

    mv, mi = lax.fori_loop(
        0, NUM // 16, body,
        (jnp.full((16,), jnp.inf, jnp.float32), jnp.zeros((16,), jnp.int32)),
    )
    gmin = mv
    for sft in (8, 4, 2, 1):
        tf_v[pl.ds(0, 16)] = gmin
        gmin = jnp.minimum(gmin, plsc.load_gather(tf_v, [(lanes + sft) & 15]))
    idxv = jnp.where(mv == gmin, mi, jnp.full((16,), NUM, jnp.int32))
    for sft in (8, 4, 2, 1):
        ti_v[pl.ds(0, 16)] = idxv
        idxv = jnp.minimum(idxv, plsc.load_gather(ti_v, [(lanes + sft) & 15]))
    start = idxv + W
    for cdx in range(DEC // 16):
        pos = start + lanes + cdx * 16
        o_v[pl.ds(cdx * 16, 16)] = plsc.load_gather(x_v, [pos])
    pltpu.sync_copy(o_v, out_hbm.at[pl.ds(pl.multiple_of(b * DEC, 8), DEC)])


def kernel(feats_in, X_in, feats_out):
    x = X_in[:, :, 0]
    dists = pl.pallas_call(
        _tc_body,
        in_specs=[pl.BlockSpec((B, T), lambda: (0, 0))],
        out_specs=pl.BlockSpec((B, PADNUM), lambda: (0, 0)),
        out_shape=jax.ShapeDtypeStruct((B, PADNUM), jnp.float32),
        scratch_shapes=[pltpu.VMEM((B, W * 128), jnp.float32)],
    )(x)
    out = _sc_argmin_gather(dists.reshape(-1), x.reshape(-1))
    return out.reshape(B, DEC, 1)


# speedup vs baseline: 1.0494x; 1.0494x over previous
import functools

import jax
import jax.numpy as jnp
from jax import lax
from jax.experimental import pallas as pl
from jax.experimental.pallas import tpu as pltpu
from jax.experimental.pallas import tpu_sc as plsc

DEC = 96
W = 192
T = 4096
B = 32
NUM = T - 2 * W
PADNUM = 3840
G = 5
NGROUP = PADNUM // (G * 128)
TILEW = (G + 1) * 128

_NC = 2
_NS = 16


def _tc_body(x_ref, idx_ref, dists_ref, kb_ref):
    for j in range(W):
        col = x_ref[:, T - W + j : T - W + j + 1]
        kb_ref[:, j * 128 : (j + 1) * 128] = jnp.broadcast_to(col, (B, 128))

    for g in range(NGROUP):
        base = g * G * 128
        accs = [jnp.zeros((B, 128), jnp.float32) for _ in range(G)]
        for jh, njl in ((0, 128), (1, 64)):
            tile = x_ref[:, pl.ds(base + 128 * jh, TILEW)]

            def body(jl, accs, tile=tile, jh=jh):
                sl = (TILEW - jl) % TILEW
                rolled = pltpu.roll(tile, sl, axis=1)
                kjb = kb_ref[:, pl.ds((128 * jh + jl) * 128, 128)]
                out = []
                for s in range(G):
                    d = rolled[:, s * 128 : (s + 1) * 128] - kjb
                    out.append(accs[s] + d * d)
                return out

            accs = jax.lax.fori_loop(0, njl, body, accs, unroll=4)
        for s in range(G):
            dists_ref[:, base + s * 128 : base + (s + 1) * 128] = accs[s] / W

    dists = dists_ref[:, :NUM]
    m = jnp.min(dists, axis=1, keepdims=True)
    iota = jax.lax.broadcasted_iota(jnp.int32, (B, NUM), 1)
    idx = jnp.min(jnp.where(dists == m, iota, NUM), axis=1, keepdims=True)
    idx_ref[:, :] = jnp.broadcast_to(idx, (B, 128))


_sc_mesh = plsc.VectorSubcoreMesh(core_axis_name="c", subcore_axis_name="s")


@functools.partial(
    pl.kernel,
    out_type=jax.ShapeDtypeStruct((B * DEC,), jnp.float32),
    mesh=_sc_mesh,
    scratch_types=[
        pltpu.VMEM((16,), jnp.int32),
        pltpu.VMEM((T,), jnp.float32),
        pltpu.VMEM((DEC,), jnp.float32),
    ],
    compiler_params=pltpu.CompilerParams(needs_layout_passes=False),
)
def _sc_gather(idx_hbm, x_hbm, out_hbm, i_v, x_v, o_v):
    b = lax.axis_index("s") * _NC + lax.axis_index("c")
    pltpu.sync_copy(idx_hbm.at[pl.ds(pl.multiple_of(b * 128, 8), 16)], i_v)
    pltpu.sync_copy(x_hbm.at[pl.ds(pl.multiple_of(b * T, 8), T)], x_v)
    lanes = lax.iota(jnp.int32, 16)
    start = i_v[pl.ds(0, 16)] + W
    for cdx in range(DEC // 16):
        pos = start + lanes + cdx * 16
        o_v[pl.ds(cdx * 16, 16)] = plsc.load_gather(x_v, [pos])
    pltpu.sync_copy(o_v, out_hbm.at[pl.ds(pl.multiple_of(b * DEC, 8), DEC)])


def kernel(feats_in, X_in, feats_out):
    x = X_in[:, :, 0]
    idx = pl.pallas_call(
        _tc_body,
        in_specs=[pl.BlockSpec((B, T), lambda: (0, 0))],
        out_specs=pl.BlockSpec((B, 128), lambda: (0, 0)),
        out_shape=jax.ShapeDtypeStruct((B, 128), jnp.int32),
        scratch_shapes=[
            pltpu.VMEM((B, PADNUM), jnp.float32),
            pltpu.VMEM((B, W * 128), jnp.float32),
        ],
    )(x)
    out = _sc_gather(idx.reshape(-1), x.reshape(-1))
    return out.reshape(B, DEC, 1)


# speedup vs baseline: 1.0707x; 1.0203x over previous
import functools

import jax
import jax.numpy as jnp
from jax import lax
from jax.experimental import pallas as pl
from jax.experimental.pallas import tpu as pltpu
from jax.experimental.pallas import tpu_sc as plsc

DEC = 96
W = 192
T = 4096
B = 32
NUM = T - 2 * W
PADNUM = 3840
G = 5
NGROUP = PADNUM // (G * 128)
TILEW = (G + 1) * 128

_NC = 2
_NS = 16


def _tc_body(x_ref, idx_ref, dists_ref, kb_ref):
    for j in range(W):
        col = x_ref[:, T - W + j : T - W + j + 1]
        kb_ref[:, j * 128 : (j + 1) * 128] = jnp.broadcast_to(col, (B, 128))

    for g in range(NGROUP):
        base = g * G * 128
        k0 = kb_ref[:, 0:128]
        accs = []
        for s in range(G):
            d = x_ref[:, pl.ds(base + s * 128, 128)] - k0
            accs.append(d * d)
        for jh, njl in ((0, 128), (1, 63)):
            tile = x_ref[:, pl.ds(base + 128 * jh, TILEW)]

            def body(jl, accs, tile=tile, jh=jh):
                jl1 = jl + 1
                rolled = pltpu.roll(tile, TILEW - jl1, axis=1)
                kjb = kb_ref[:, pl.ds((128 * jh + jl1) * 128, 128)]
                out = []
                for s in range(G):
                    d = rolled[:, s * 128 : (s + 1) * 128] - kjb
                    out.append(accs[s] + d * d)
                return out

            accs = jax.lax.fori_loop(0, njl, body, accs, unroll=4)
        for s in range(G):
            dists_ref[:, base + s * 128 : base + (s + 1) * 128] = accs[s] / W

    dists = dists_ref[:, :NUM]
    m = jnp.min(dists, axis=1, keepdims=True)
    iota = jax.lax.broadcasted_iota(jnp.int32, (B, NUM), 1)
    idx = jnp.min(jnp.where(dists == m, iota, NUM), axis=1, keepdims=True)
    idx_ref[:, :] = jnp.broadcast_to(idx, (B, 128))


_sc_mesh = plsc.VectorSubcoreMesh(core_axis_name="c", subcore_axis_name="s")


@functools.partial(
    pl.kernel,
    out_type=jax.ShapeDtypeStruct((B * DEC,), jnp.float32),
    mesh=_sc_mesh,
    scratch_types=[
        pltpu.VMEM((16,), jnp.int32),
        pltpu.VMEM((T,), jnp.float32),
        pltpu.VMEM((DEC,), jnp.float32),
    ],
    compiler_params=pltpu.CompilerParams(needs_layout_passes=False),
)
def _sc_gather(idx_hbm, x_hbm, out_hbm, i_v, x_v, o_v):
    b = lax.axis_index("s") * _NC + lax.axis_index("c")
    pltpu.sync_copy(idx_hbm.at[pl.ds(pl.multiple_of(b * 128, 8), 16)], i_v)
    pltpu.sync_copy(x_hbm.at[pl.ds(pl.multiple_of(b * T, 8), T)], x_v)
    lanes = lax.iota(jnp.int32, 16)
    start = i_v[pl.ds(0, 16)] + W
    for cdx in range(DEC // 16):
        pos = start + lanes + cdx * 16
        o_v[pl.ds(cdx * 16, 16)] = plsc.load_gather(x_v, [pos])
    pltpu.sync_copy(o_v, out_hbm.at[pl.ds(pl.multiple_of(b * DEC, 8), DEC)])


def kernel(feats_in, X_in, feats_out):
    x = X_in[:, :, 0]
    idx = pl.pallas_call(
        _tc_body,
        in_specs=[pl.BlockSpec((B, T), lambda: (0, 0))],
        out_specs=pl.BlockSpec((B, 128), lambda: (0, 0)),
        out_shape=jax.ShapeDtypeStruct((B, 128), jnp.int32),
        scratch_shapes=[
            pltpu.VMEM((B, PADNUM), jnp.float32),
            pltpu.VMEM((B, W * 128), jnp.float32),
        ],
    )(x)
    out = _sc_gather(idx.reshape(-1), x.reshape(-1))
    return out.reshape(B, DEC, 1)


# speedup vs baseline: 1.2731x; 1.1890x over previous
import functools

import jax
import jax.numpy as jnp
from jax import lax
from jax.experimental import pallas as pl
from jax.experimental.pallas import tpu as pltpu
from jax.experimental.pallas import tpu_sc as plsc

DEC = 96
W = 192
T = 4096
B = 32
NUM = T - 2 * W
PADNUM = 3840
G2 = 10
NG2 = PADNUM // (G2 * 128)
TILE2 = G2 * 128 + 256
TILE1 = G2 * 128 + 128

_NC = 2
_NS = 16


def _tc_body(x_ref, idx_ref, dists_ref, kb_ref):
    for j in range(W):
        col = x_ref[:, T - W + j : T - W + j + 1]
        kb_ref[:, j * 128 : (j + 1) * 128] = jnp.broadcast_to(col, (B, 128))

    for g in range(NG2):
        base = g * G2 * 128
        k0 = kb_ref[:, 0:128]
        for s in range(G2):
            d = x_ref[:, pl.ds(base + s * 128, 128)] - k0
            dists_ref[:, base + s * 128 : base + (s + 1) * 128] = d * d

        tile2 = x_ref[:, pl.ds(base, TILE2)]

        def body2(jl, _, base=base, tile2=tile2):
            jl1 = jl + 1
            rolled = pltpu.roll(tile2, TILE2 - jl1, axis=1)
            kb1 = kb_ref[:, pl.ds(jl1 * 128, 128)]
            kb2 = kb_ref[:, pl.ds((jl1 + 128) * 128, 128)]
            for s in range(G2):
                d1 = rolled[:, s * 128 : (s + 1) * 128] - kb1
                d2 = rolled[:, (s + 1) * 128 : (s + 2) * 128] - kb2
                cur = dists_ref[:, base + s * 128 : base + (s + 1) * 128]
                dists_ref[:, base + s * 128 : base + (s + 1) * 128] = (
                    cur + d1 * d1 + d2 * d2)
            return 0

        jax.lax.fori_loop(0, 63, body2, 0, unroll=2)

        tile1 = x_ref[:, pl.ds(base, TILE1)]

        def body1(jl, _, base=base, tile1=tile1):
            jl1 = jl + 1
            rolled = pltpu.roll(tile1, TILE1 - jl1, axis=1)
            kb1 = kb_ref[:, pl.ds(jl1 * 128, 128)]
            for s in range(G2):
                d1 = rolled[:, s * 128 : (s + 1) * 128] - kb1
                cur = dists_ref[:, base + s * 128 : base + (s + 1) * 128]
                dists_ref[:, base + s * 128 : base + (s + 1) * 128] = (
                    cur + d1 * d1)
            return 0

        jax.lax.fori_loop(63, 128, body1, 0, unroll=2)

    dists = dists_ref[:, :NUM] / W
    m = jnp.min(dists, axis=1, keepdims=True)
    iota = jax.lax.broadcasted_iota(jnp.int32, (B, NUM), 1)
    idx = jnp.min(jnp.where(dists == m, iota, NUM), axis=1, keepdims=True)
    idx_ref[:, :] = jnp.broadcast_to(idx, (B, 128))


_sc_mesh = plsc.VectorSubcoreMesh(core_axis_name="c", subcore_axis_name="s")


@functools.partial(
    pl.kernel,
    out_type=jax.ShapeDtypeStruct((B * DEC,), jnp.float32),
    mesh=_sc_mesh,
    scratch_types=[
        pltpu.VMEM((16,), jnp.int32),
        pltpu.VMEM((T,), jnp.float32),
        pltpu.VMEM((DEC,), jnp.float32),
    ],
    compiler_params=pltpu.CompilerParams(needs_layout_passes=False),
)
def _sc_gather(idx_hbm, x_hbm, out_hbm, i_v, x_v, o_v):
    b = lax.axis_index("s") * _NC + lax.axis_index("c")
    pltpu.sync_copy(idx_hbm.at[pl.ds(pl.multiple_of(b * 128, 8), 16)], i_v)
    pltpu.sync_copy(x_hbm.at[pl.ds(pl.multiple_of(b * T, 8), T)], x_v)
    lanes = lax.iota(jnp.int32, 16)
    start = i_v[pl.ds(0, 16)] + W
    for cdx in range(DEC // 16):
        pos = start + lanes + cdx * 16
        o_v[pl.ds(cdx * 16, 16)] = plsc.load_gather(x_v, [pos])
    pltpu.sync_copy(o_v, out_hbm.at[pl.ds(pl.multiple_of(b * DEC, 8), DEC)])


def kernel(feats_in, X_in, feats_out):
    x = X_in[:, :, 0]
    idx = pl.pallas_call(
        _tc_body,
        in_specs=[pl.BlockSpec((B, T), lambda: (0, 0))],
        out_specs=pl.BlockSpec((B, 128), lambda: (0, 0)),
        out_shape=jax.ShapeDtypeStruct((B, 128), jnp.int32),
        scratch_shapes=[
            pltpu.VMEM((B, PADNUM), jnp.float32),
            pltpu.VMEM((B, W * 128), jnp.float32),
        ],
    )(x)
    out = _sc_gather(idx.reshape(-1), x.reshape(-1))
    return out.reshape(B, DEC, 1)


# speedup vs baseline: 1.4741x; 1.1579x over previous
import functools

import jax
import jax.numpy as jnp
from jax import lax
from jax.experimental import pallas as pl
from jax.experimental.pallas import tpu as pltpu
from jax.experimental.pallas import tpu_sc as plsc

DEC = 96
W = 192
T = 4096
B = 32
NUM = T - 2 * W
PADNUM = 3840
G2 = 30
NG2 = PADNUM // (G2 * 128)
TILE2 = G2 * 128 + 256
TILE1 = G2 * 128 + 128

_NC = 2
_NS = 16


def _tc_body(x_ref, idx_ref, dists_ref, kb_ref):
    for j in range(W):
        col = x_ref[:, T - W + j : T - W + j + 1]
        kb_ref[:, j * 128 : (j + 1) * 128] = jnp.broadcast_to(col, (B, 128))

    for g in range(NG2):
        base = g * G2 * 128
        k0 = kb_ref[:, 0:128]
        for s in range(G2):
            d = x_ref[:, pl.ds(base + s * 128, 128)] - k0
            dists_ref[:, base + s * 128 : base + (s + 1) * 128] = d * d

        tile2 = x_ref[:, pl.ds(base, TILE2)]

        def body2(jl, _, base=base, tile2=tile2):
            jl1 = jl + 1
            rolled = pltpu.roll(tile2, TILE2 - jl1, axis=1)
            kb1 = kb_ref[:, pl.ds(jl1 * 128, 128)]
            kb2 = kb_ref[:, pl.ds((jl1 + 128) * 128, 128)]
            for s in range(G2):
                d1 = rolled[:, s * 128 : (s + 1) * 128] - kb1
                d2 = rolled[:, (s + 1) * 128 : (s + 2) * 128] - kb2
                cur = dists_ref[:, base + s * 128 : base + (s + 1) * 128]
                dists_ref[:, base + s * 128 : base + (s + 1) * 128] = (
                    cur + d1 * d1 + d2 * d2)
            return 0

        jax.lax.fori_loop(0, 63, body2, 0, unroll=2)

        tile1 = x_ref[:, pl.ds(base, TILE1)]

        def body1(jl, _, base=base, tile1=tile1):
            jl1 = jl + 1
            rolled = pltpu.roll(tile1, TILE1 - jl1, axis=1)
            kb1 = kb_ref[:, pl.ds(jl1 * 128, 128)]
            for s in range(G2):
                d1 = rolled[:, s * 128 : (s + 1) * 128] - kb1
                cur = dists_ref[:, base + s * 128 : base + (s + 1) * 128]
                dists_ref[:, base + s * 128 : base + (s + 1) * 128] = (
                    cur + d1 * d1)
            return 0

        jax.lax.fori_loop(63, 128, body1, 0, unroll=2)

    dists = dists_ref[:, :NUM] / W
    m = jnp.min(dists, axis=1, keepdims=True)
    iota = jax.lax.broadcasted_iota(jnp.int32, (B, NUM), 1)
    idx = jnp.min(jnp.where(dists == m, iota, NUM), axis=1, keepdims=True)
    idx_ref[:, :] = jnp.broadcast_to(idx, (B, 128))


_sc_mesh = plsc.VectorSubcoreMesh(core_axis_name="c", subcore_axis_name="s")


@functools.partial(
    pl.kernel,
    out_type=jax.ShapeDtypeStruct((B * DEC,), jnp.float32),
    mesh=_sc_mesh,
    scratch_types=[
        pltpu.VMEM((16,), jnp.int32),
        pltpu.VMEM((T,), jnp.float32),
        pltpu.VMEM((DEC,), jnp.float32),
    ],
    compiler_params=pltpu.CompilerParams(needs_layout_passes=False),
)
def _sc_gather(idx_hbm, x_hbm, out_hbm, i_v, x_v, o_v):
    b = lax.axis_index("s") * _NC + lax.axis_index("c")
    pltpu.sync_copy(idx_hbm.at[pl.ds(pl.multiple_of(b * 128, 8), 16)], i_v)
    pltpu.sync_copy(x_hbm.at[pl.ds(pl.multiple_of(b * T, 8), T)], x_v)
    lanes = lax.iota(jnp.int32, 16)
    start = i_v[pl.ds(0, 16)] + W
    for cdx in range(DEC // 16):
        pos = start + lanes + cdx * 16
        o_v[pl.ds(cdx * 16, 16)] = plsc.load_gather(x_v, [pos])
    pltpu.sync_copy(o_v, out_hbm.at[pl.ds(pl.multiple_of(b * DEC, 8), DEC)])


def kernel(feats_in, X_in, feats_out):
    x = X_in[:, :, 0]
    idx = pl.pallas_call(
        _tc_body,
        in_specs=[pl.BlockSpec((B, T), lambda: (0, 0))],
        out_specs=pl.BlockSpec((B, 128), lambda: (0, 0)),
        out_shape=jax.ShapeDtypeStruct((B, 128), jnp.int32),
        scratch_shapes=[
            pltpu.VMEM((B, PADNUM), jnp.float32),
            pltpu.VMEM((B, W * 128), jnp.float32),
        ],
    )(x)
    out = _sc_gather(idx.reshape(-1), x.reshape(-1))
    return out.reshape(B, DEC, 1)


# speedup vs baseline: 1.5984x; 1.0843x over previous
import functools

import jax
import jax.numpy as jnp
from jax import lax
from jax.experimental import pallas as pl
from jax.experimental.pallas import tpu as pltpu
from jax.experimental.pallas import tpu_sc as plsc

DEC = 96
W = 192
T = 4096
B = 32
NUM = T - 2 * W
PADNUM = 3840
G2 = 30
NG2 = PADNUM // (G2 * 128)
TILE2 = G2 * 128 + 256
TILE1 = G2 * 128 + 128

_NC = 2
_NS = 16


def _tc_body(x_ref, idx_ref, dists_ref, kb_ref):
    for j in range(W):
        col = x_ref[:, T - W + j : T - W + j + 1]
        kb_ref[:, j * 128 : (j + 1) * 128] = jnp.broadcast_to(col, (B, 128))

    for g in range(NG2):
        base = g * G2 * 128
        k0 = kb_ref[:, 0:128]
        for s in range(G2):
            d = x_ref[:, pl.ds(base + s * 128, 128)] - k0
            dists_ref[:, base + s * 128 : base + (s + 1) * 128] = d * d

        tile2 = x_ref[:, pl.ds(base, TILE2)]

        def body2(jl, _, base=base, tile2=tile2):
            jl1 = jl + 1
            rolled = pltpu.roll(tile2, TILE2 - jl1, axis=1)
            kb1 = kb_ref[:, pl.ds(jl1 * 128, 128)]
            kb2 = kb_ref[:, pl.ds((jl1 + 128) * 128, 128)]
            for s in range(G2):
                d1 = rolled[:, s * 128 : (s + 1) * 128] - kb1
                d2 = rolled[:, (s + 1) * 128 : (s + 2) * 128] - kb2
                cur = dists_ref[:, base + s * 128 : base + (s + 1) * 128]
                dists_ref[:, base + s * 128 : base + (s + 1) * 128] = (
                    cur + d1 * d1 + d2 * d2)
            return 0

        jax.lax.fori_loop(0, 63, body2, 0, unroll=4)

        tile1 = x_ref[:, pl.ds(base, TILE1)]

        def body1(jl, _, base=base, tile1=tile1):
            jl1 = jl + 1
            rolled = pltpu.roll(tile1, TILE1 - jl1, axis=1)
            kb1 = kb_ref[:, pl.ds(jl1 * 128, 128)]
            for s in range(G2):
                d1 = rolled[:, s * 128 : (s + 1) * 128] - kb1
                cur = dists_ref[:, base + s * 128 : base + (s + 1) * 128]
                dists_ref[:, base + s * 128 : base + (s + 1) * 128] = (
                    cur + d1 * d1)
            return 0

        jax.lax.fori_loop(63, 128, body1, 0, unroll=4)

    dists = dists_ref[:, :NUM] / W
    m = jnp.min(dists, axis=1, keepdims=True)
    iota = jax.lax.broadcasted_iota(jnp.int32, (B, NUM), 1)
    idx = jnp.min(jnp.where(dists == m, iota, NUM), axis=1, keepdims=True)
    idx_ref[:, :] = jnp.broadcast_to(idx, (B, 128))


_sc_mesh = plsc.VectorSubcoreMesh(core_axis_name="c", subcore_axis_name="s")


@functools.partial(
    pl.kernel,
    out_type=jax.ShapeDtypeStruct((B * DEC,), jnp.float32),
    mesh=_sc_mesh,
    scratch_types=[
        pltpu.VMEM((16,), jnp.int32),
        pltpu.VMEM((T,), jnp.float32),
        pltpu.VMEM((DEC,), jnp.float32),
    ],
    compiler_params=pltpu.CompilerParams(needs_layout_passes=False),
)
def _sc_gather(idx_hbm, x_hbm, out_hbm, i_v, x_v, o_v):
    b = lax.axis_index("s") * _NC + lax.axis_index("c")
    pltpu.sync_copy(idx_hbm.at[pl.ds(pl.multiple_of(b * 128, 8), 16)], i_v)
    pltpu.sync_copy(x_hbm.at[pl.ds(pl.multiple_of(b * T, 8), T)], x_v)
    lanes = lax.iota(jnp.int32, 16)
    start = i_v[pl.ds(0, 16)] + W
    for cdx in range(DEC // 16):
        pos = start + lanes + cdx * 16
        o_v[pl.ds(cdx * 16, 16)] = plsc.load_gather(x_v, [pos])
    pltpu.sync_copy(o_v, out_hbm.at[pl.ds(pl.multiple_of(b * DEC, 8), DEC)])


def kernel(feats_in, X_in, feats_out):
    x = X_in[:, :, 0]
    idx = pl.pallas_call(
        _tc_body,
        in_specs=[pl.BlockSpec((B, T), lambda: (0, 0))],
        out_specs=pl.BlockSpec((B, 128), lambda: (0, 0)),
        out_shape=jax.ShapeDtypeStruct((B, 128), jnp.int32),
        scratch_shapes=[
            pltpu.VMEM((B, PADNUM), jnp.float32),
            pltpu.VMEM((B, W * 128), jnp.float32),
        ],
    )(x)
    out = _sc_gather(idx.reshape(-1), x.reshape(-1))
    return out.reshape(B, DEC, 1)


# speedup vs baseline: 1.6720x; 1.0461x over previous
import functools

import jax
import jax.numpy as jnp
from jax import lax
from jax.experimental import pallas as pl
from jax.experimental.pallas import tpu as pltpu
from jax.experimental.pallas import tpu_sc as plsc

DEC = 96
W = 192
T = 4096
B = 32
NUM = T - 2 * W
PADNUM = 3840
G2 = 30
NG2 = PADNUM // (G2 * 128)
TILE2 = G2 * 128 + 256
TILE1 = G2 * 128 + 128

_NC = 2
_NS = 16


def _tc_body(x_ref, idx_ref, dists_ref, kb_ref):
    for j in range(W):
        col = x_ref[:, T - W + j : T - W + j + 1]
        kb_ref[:, j * 128 : (j + 1) * 128] = jnp.broadcast_to(col, (B, 128))

    for g in range(NG2):
        base = g * G2 * 128
        k0 = kb_ref[:, 0:128]
        for s in range(G2):
            d = x_ref[:, pl.ds(base + s * 128, 128)] - k0
            dists_ref[:, base + s * 128 : base + (s + 1) * 128] = d * d

        tile2 = x_ref[:, pl.ds(base, TILE2)]

        def body2(jl, _, base=base, tile2=tile2):
            jl1 = jl + 1
            rolled = pltpu.roll(tile2, TILE2 - jl1, axis=1)
            kb1 = kb_ref[:, pl.ds(jl1 * 128, 128)]
            kb2 = kb_ref[:, pl.ds((jl1 + 128) * 128, 128)]
            for s in range(G2):
                d1 = rolled[:, s * 128 : (s + 1) * 128] - kb1
                d2 = rolled[:, (s + 1) * 128 : (s + 2) * 128] - kb2
                cur = dists_ref[:, base + s * 128 : base + (s + 1) * 128]
                dists_ref[:, base + s * 128 : base + (s + 1) * 128] = (
                    cur + d1 * d1 + d2 * d2)
            return 0

        jax.lax.fori_loop(0, 63, body2, 0, unroll=8)

        tile1 = x_ref[:, pl.ds(base, TILE1)]

        def body1(jl, _, base=base, tile1=tile1):
            jl1 = jl + 1
            rolled = pltpu.roll(tile1, TILE1 - jl1, axis=1)
            kb1 = kb_ref[:, pl.ds(jl1 * 128, 128)]
            for s in range(G2):
                d1 = rolled[:, s * 128 : (s + 1) * 128] - kb1
                cur = dists_ref[:, base + s * 128 : base + (s + 1) * 128]
                dists_ref[:, base + s * 128 : base + (s + 1) * 128] = (
                    cur + d1 * d1)
            return 0

        jax.lax.fori_loop(63, 128, body1, 0, unroll=8)

    dists = dists_ref[:, :NUM] / W
    m = jnp.min(dists, axis=1, keepdims=True)
    iota = jax.lax.broadcasted_iota(jnp.int32, (B, NUM), 1)
    idx = jnp.min(jnp.where(dists == m, iota, NUM), axis=1, keepdims=True)
    idx_ref[:, :] = jnp.broadcast_to(idx, (B, 128))


_sc_mesh = plsc.VectorSubcoreMesh(core_axis_name="c", subcore_axis_name="s")


@functools.partial(
    pl.kernel,
    out_type=jax.ShapeDtypeStruct((B * DEC,), jnp.float32),
    mesh=_sc_mesh,
    scratch_types=[
        pltpu.VMEM((16,), jnp.int32),
        pltpu.VMEM((T,), jnp.float32),
        pltpu.VMEM((DEC,), jnp.float32),
    ],
    compiler_params=pltpu.CompilerParams(needs_layout_passes=False),
)
def _sc_gather(idx_hbm, x_hbm, out_hbm, i_v, x_v, o_v):
    b = lax.axis_index("s") * _NC + lax.axis_index("c")
    pltpu.sync_copy(idx_hbm.at[pl.ds(pl.multiple_of(b * 128, 8), 16)], i_v)
    pltpu.sync_copy(x_hbm.at[pl.ds(pl.multiple_of(b * T, 8), T)], x_v)
    lanes = lax.iota(jnp.int32, 16)
    start = i_v[pl.ds(0, 16)] + W
    for cdx in range(DEC // 16):
        pos = start + lanes + cdx * 16
        o_v[pl.ds(cdx * 16, 16)] = plsc.load_gather(x_v, [pos])
    pltpu.sync_copy(o_v, out_hbm.at[pl.ds(pl.multiple_of(b * DEC, 8), DEC)])


def kernel(feats_in, X_in, feats_out):
    x = X_in[:, :, 0]
    idx = pl.pallas_call(
        _tc_body,
        in_specs=[pl.BlockSpec((B, T), lambda: (0, 0))],
        out_specs=pl.BlockSpec((B, 128), lambda: (0, 0)),
        out_shape=jax.ShapeDtypeStruct((B, 128), jnp.int32),
        scratch_shapes=[
            pltpu.VMEM((B, PADNUM), jnp.float32),
            pltpu.VMEM((B, W * 128), jnp.float32),
        ],
    )(x)
    out = _sc_gather(idx.reshape(-1), x.reshape(-1))
    return out.reshape(B, DEC, 1)
